# baseline (speedup 1.0000x reference)
import jax
import jax.numpy as jnp
from jax.experimental import pallas as pl

TOKEN_BLOCK = 1024


def _gate_block(x_ref, wt_ref, o_ref):
    logits = jax.lax.dot_general(
        x_ref[...],
        wt_ref[...],
        dimension_numbers=(((1,), (0,)), ((), ())),
        preferred_element_type=jnp.float32,
        precision=jax.lax.Precision.DEFAULT,
    )
    mask = (logits > 0.0).astype(jnp.int32)
    o_ref[...] = mask.T


@jax.jit
def kernel(cls_hidden_states, gate_w):
    tokens, hidden = cls_hidden_states.shape
    num_experts = gate_w.shape[0]
    wt = gate_w.T

    grid = (tokens // TOKEN_BLOCK,)
    mask_t = pl.pallas_call(
        _gate_block,
        grid=grid,
        in_specs=[
            pl.BlockSpec((TOKEN_BLOCK, hidden), lambda i: (i, 0)),
            pl.BlockSpec((hidden, num_experts), lambda i: (0, 0)),
        ],
        out_specs=pl.BlockSpec((num_experts, TOKEN_BLOCK), lambda i: (0, i)),
        out_shape=jax.ShapeDtypeStruct((num_experts, tokens), jnp.int32),
    )(cls_hidden_states, wt)
    return mask_t.T


# speedup vs baseline: 1.0967x; 1.0967x over previous
import jax
import jax.numpy as jnp
from jax.experimental import pallas as pl

TOKEN_BLOCK = 1024


def _gate_block(w_ref, x_ref, o_ref):
    logits_t = jax.lax.dot_general(
        w_ref[...],
        x_ref[...],
        dimension_numbers=(((1,), (1,)), ((), ())),
        preferred_element_type=jnp.float32,
        precision=jax.lax.Precision.DEFAULT,
    )
    o_ref[...] = (logits_t > 0.0).astype(jnp.int32)


@jax.jit
def kernel(cls_hidden_states, gate_w):
    tokens, hidden = cls_hidden_states.shape
    num_experts = gate_w.shape[0]

    grid = (tokens // TOKEN_BLOCK,)
    mask_t = pl.pallas_call(
        _gate_block,
        grid=grid,
        in_specs=[
            pl.BlockSpec((num_experts, hidden), lambda i: (0, 0)),
            pl.BlockSpec((TOKEN_BLOCK, hidden), lambda i: (i, 0)),
        ],
        out_specs=pl.BlockSpec((num_experts, TOKEN_BLOCK), lambda i: (0, i)),
        out_shape=jax.ShapeDtypeStruct((num_experts, tokens), jnp.int32),
    )(gate_w, cls_hidden_states)
    return mask_t.T
